# baseline (speedup 1.0000x reference)
import functools

import jax
import jax.numpy as jnp
from jax import lax
from jax.experimental import pallas as pl
from jax.experimental.pallas import tpu as pltpu
from jax.experimental.pallas import tpu_sc as plsc

_N = 10000
_F = 128
_H = 64
_E = 320000
_NC = 2
_NS = 16
_NW = _NC * _NS
_CH = 128
_K = 80
_EPT = _K * _CH
_EP = _NW * _EPT
_TS = 640
_NP = _NS * _TS
_R = 2000
_G = _N // _R

_mesh = plsc.VectorSubcoreMesh(core_axis_name="c", subcore_axis_name="s")


@functools.partial(
    pl.kernel,
    out_type=jax.ShapeDtypeStruct((_NC, _NP), jnp.float32),
    mesh=_mesh,
    compiler_params=pltpu.CompilerParams(use_tc_tiling_on_sc=False),
    scratch_types=[
        pltpu.VMEM((_K, _CH), jnp.int32),
        pltpu.VMEM((_K, _CH), jnp.float32),
        pltpu.VMEM((_TS,), jnp.float32),
        pltpu.VMEM_SHARED((_NP,), jnp.float32),
    ],
)
def _sc_deg(dst_hbm, ew_hbm, out_hbm, idx_v, ew_v, z_v, deg_sh):
    cid = lax.axis_index("c")
    sid = lax.axis_index("s")
    wid = sid * _NC + cid
    zero = jnp.zeros((16,), jnp.float32)
    for i in range(_TS // 16):
        z_v[pl.ds(i * 16, 16)] = zero
    pltpu.sync_copy(z_v, deg_sh.at[pl.ds(sid * _TS, _TS)])
    pltpu.sync_copy(dst_hbm.at[wid], idx_v)
    pltpu.sync_copy(ew_hbm.at[wid], ew_v)
    plsc.subcore_barrier()

    def body(j, carry):
        pltpu.sync_copy(ew_v.at[j], deg_sh.at[idx_v.at[j]], add=True)
        return carry

    lax.fori_loop(0, _K, body, 0)
    plsc.subcore_barrier()
    pltpu.sync_copy(deg_sh.at[pl.ds(sid * _TS, _TS)],
                    out_hbm.at[cid, pl.ds(sid * _TS, _TS)])


@functools.partial(
    pl.kernel,
    out_type=jax.ShapeDtypeStruct((_NC, _NP, _H), jnp.float32),
    mesh=_mesh,
    compiler_params=pltpu.CompilerParams(use_tc_tiling_on_sc=False,
                                         needs_layout_passes=False),
    scratch_types=[
        pltpu.VMEM((_K, _CH), jnp.int32),
        pltpu.VMEM((_K, _CH), jnp.int32),
        pltpu.VMEM((_K, _CH), jnp.float32),
        pltpu.VMEM((_CH, _H // 2), jnp.int32),
        pltpu.VMEM((_CH, _H // 2), jnp.int32),
        pltpu.VMEM((_CH, _H // 2), jnp.int32),
        pltpu.VMEM((_CH, _H // 2), jnp.int32),
        pltpu.VMEM((_CH, _H // 2), jnp.int32),
        pltpu.VMEM((_CH, _H // 2), jnp.int32),
        pltpu.VMEM((_CH, _H // 2), jnp.int32),
        pltpu.VMEM((_CH, _H // 2), jnp.int32),
        pltpu.VMEM((_CH, _H), jnp.float32),
        pltpu.VMEM((_CH, _H), jnp.float32),
        pltpu.VMEM((_CH, _H), jnp.float32),
        pltpu.VMEM_SHARED((_NP, _H), jnp.float32),
        pltpu.SemaphoreType.DMA,
        pltpu.SemaphoreType.DMA,
        pltpu.SemaphoreType.DMA,
        pltpu.SemaphoreType.DMA,
        pltpu.SemaphoreType.DMA,
        pltpu.SemaphoreType.DMA,
        pltpu.SemaphoreType.DMA,
        pltpu.SemaphoreType.DMA,
        pltpu.SemaphoreType.DMA,
        pltpu.SemaphoreType.DMA,
        pltpu.SemaphoreType.DMA,
    ],
)
def _sc_conv(h_hbm, src_hbm, dst_hbm, ew_hbm, out_hbm,
             src_v, dst_v, ew_v,
             gb0, gb1, gb2, gb3, gb4, gb5, gb6, gb7,
             sb0, sb1, sb2,
             acc_sh,
             g0, g1, g2, g3, g4, g5, g6, g7, s0, s1, s2):
    cid = lax.axis_index("c")
    sid = lax.axis_index("s")
    wid = sid * _NC + cid
    zero = jnp.zeros((16,), jnp.float32)
    for i in range(_CH):
        for f in range(_H // 16):
            sb0[i, pl.ds(f * 16, 16)] = zero
    for t in range(_TS // _CH):
        pltpu.sync_copy(sb0, acc_sh.at[pl.ds(sid * _TS + t * _CH, _CH)])
    pltpu.sync_copy(src_hbm.at[wid], src_v)
    pltpu.sync_copy(dst_hbm.at[wid], dst_v)
    pltpu.sync_copy(ew_hbm.at[wid], ew_v)
    plsc.subcore_barrier()

    _MASK = jnp.int32(-65536)

    def _mult(j, gb, sb):
        def edge_group(g, c2):
            wv = ew_v[j, pl.ds(g * 16, 16)]
            for l in range(16):
                e = g * 16 + l
                w = wv[l]
                for f in range(2):
                    wi = gb[e, pl.ds(f * 16, 16)]
                    ev = plsc.bitcast(wi << 16, jnp.float32)
                    od = plsc.bitcast(wi & _MASK, jnp.float32)
                    sb[e, pl.ds(f * 32, 16)] = ev * w
                    sb[e, pl.ds(f * 32 + 16, 16)] = od * w
            return c2
        lax.fori_loop(0, _CH // 16, edge_group, 0, unroll=2)

    def _g_start(j, gb, sem):
        pltpu.async_copy(h_hbm.at[src_v.at[j]], gb, sem)

    def _g_wait(gb, sem):
        pltpu.make_async_copy(h_hbm.at[src_v.at[0]], gb, sem).wait()

    def _s_start(j, sb, sem):
        pltpu.async_copy(sb, acc_sh.at[dst_v.at[j]], sem, add=True)

    def _s_wait(sb, sem):
        pltpu.make_async_copy(sb, acc_sh.at[dst_v.at[0]], sem).wait()

    gbufs = (gb0, gb1, gb2, gb3, gb4, gb5, gb6, gb7)
    sbufs = (sb0, sb1, sb2)
    gs = (g0, g1, g2, g3, g4, g5, g6, g7)
    ss = (s0, s1, s2)
    _NG = 8
    _SBI = (0, 1, 2, 0, 1, 2, 0, 1)
    _SDIST = (2, 2, 5, 3, 3, 3, 3, 3)

    for b in range(_NG - 1):
        _g_start(b, gbufs[b], gs[b])

    def octet(i, carry):
        for b in range(_NG):
            c = i + b
            _g_wait(gbufs[b], gs[b])
            nb = (b + _NG - 1) % _NG

            @pl.when(c + _NG - 1 < _K)
            def _(c=c, nb=nb):
                _g_start(c + _NG - 1, gbufs[nb], gs[nb])

            sbi = _SBI[b]

            @pl.when(c >= _SDIST[b])
            def _(sbi=sbi):
                _s_wait(sbufs[sbi], ss[sbi])

            _mult(c, gbufs[b], sbufs[sbi])
            _s_start(c, sbufs[sbi], ss[sbi])
        return carry

    lax.fori_loop(0, _K // _NG, lambda i, c: octet(i * _NG, c), 0)
    for b in range(3):
        _s_wait(sbufs[b], ss[b])
    plsc.subcore_barrier()
    pltpu.sync_copy(acc_sh.at[pl.ds(sid * _TS, _TS)],
                    out_hbm.at[cid, pl.ds(sid * _TS, _TS)])


def _pack_rows(h, hb_ref):
    for g in range(2):
        lo = lax.bitcast_convert_type(
            h[:, 32 * g:32 * g + 16].astype(jnp.bfloat16), jnp.int16)
        hi = lax.bitcast_convert_type(
            h[:, 32 * g + 16:32 * g + 32].astype(jnp.bfloat16), jnp.int16)
        word = (lo.astype(jnp.int32) & 0xFFFF) | (hi.astype(jnp.int32) << 16)
        hb_ref[:, 16 * g:16 * g + 16] = word


def _tc_a_body(x_ref, w_ref, degT_ref, h1s_ref, dinv_ref, hb_ref):
    deg = degT_ref[:, 0:1] + degT_ref[:, 1:2] + 1.0
    dinv = jnp.where(deg > 0, lax.rsqrt(jnp.maximum(deg, 1e-30)), 0.0)
    h = jnp.dot(x_ref[...], w_ref[...], preferred_element_type=jnp.float32)
    h1s = h * dinv
    h1s_ref[...] = h1s
    dinv_ref[...] = dinv
    _pack_rows(h1s, hb_ref)


_tc_a = pl.pallas_call(
    _tc_a_body,
    grid=(_G,),
    in_specs=[
        pl.BlockSpec((_R, _F), lambda i: (i, 0)),
        pl.BlockSpec((_F, _H), lambda i: (0, 0)),
        pl.BlockSpec((_R, 2), lambda i: (i, 0)),
    ],
    out_specs=[
        pl.BlockSpec((_R, _H), lambda i: (i, 0)),
        pl.BlockSpec((_R, 1), lambda i: (i, 0)),
        pl.BlockSpec((_R, _H // 2), lambda i: (i, 0)),
    ],
    out_shape=[
        jax.ShapeDtypeStruct((_N, _H), jnp.float32),
        jax.ShapeDtypeStruct((_N, 1), jnp.float32),
        jax.ShapeDtypeStruct((_N, _H // 2), jnp.int32),
    ],
)


def _tc_stats_body(acc_ref, hs_ref, dinv_ref, b_ref, r_ref, s_ref, sq_ref):
    a = acc_ref[0] + acc_ref[1] + hs_ref[...]
    r = jnp.maximum(b_ref[...] + dinv_ref[...] * a, 0.0)
    r_ref[...] = r

    @pl.when(pl.program_id(0) == 0)
    def _():
        s_ref[...] = jnp.zeros_like(s_ref)
        sq_ref[...] = jnp.zeros_like(sq_ref)

    s_ref[...] += jnp.sum(r, axis=0, keepdims=True)
    sq_ref[...] += jnp.sum(r * r, axis=0, keepdims=True)


_tc_stats = pl.pallas_call(
    _tc_stats_body,
    grid=(_G,),
    in_specs=[
        pl.BlockSpec((_NC, _R, _H), lambda i: (0, i, 0)),
        pl.BlockSpec((_R, _H), lambda i: (i, 0)),
        pl.BlockSpec((_R, 1), lambda i: (i, 0)),
        pl.BlockSpec((1, _H), lambda i: (0, 0)),
    ],
    out_specs=[
        pl.BlockSpec((_R, _H), lambda i: (i, 0)),
        pl.BlockSpec((1, _H), lambda i: (0, 0)),
        pl.BlockSpec((1, _H), lambda i: (0, 0)),
    ],
    out_shape=[
        jax.ShapeDtypeStruct((_N, _H), jnp.float32),
        jax.ShapeDtypeStruct((1, _H), jnp.float32),
        jax.ShapeDtypeStruct((1, _H), jnp.float32),
    ],
)


def _tc_bn_body(r_ref, s_ref, sq_ref, g_ref, be_ref, dinv_ref, w_ref,
                x1_ref, hs2_ref, hb_ref):
    m = s_ref[...] * (1.0 / _N)
    v = sq_ref[...] * (1.0 / _N) - m * m
    scale = g_ref[...] * lax.rsqrt(v + 1e-5)
    x1 = (r_ref[...] - m) * scale + be_ref[...]
    x1_ref[...] = x1
    hs2 = jnp.dot(x1, w_ref[...],
                  preferred_element_type=jnp.float32) * dinv_ref[...]
    hs2_ref[...] = hs2
    _pack_rows(hs2, hb_ref)


_tc_bn = pl.pallas_call(
    _tc_bn_body,
    grid=(_G,),
    in_specs=[
        pl.BlockSpec((_R, _H), lambda i: (i, 0)),
        pl.BlockSpec((1, _H), lambda i: (0, 0)),
        pl.BlockSpec((1, _H), lambda i: (0, 0)),
        pl.BlockSpec((1, _H), lambda i: (0, 0)),
        pl.BlockSpec((1, _H), lambda i: (0, 0)),
        pl.BlockSpec((_R, 1), lambda i: (i, 0)),
        pl.BlockSpec((_H, _H), lambda i: (0, 0)),
    ],
    out_specs=[
        pl.BlockSpec((_R, _H), lambda i: (i, 0)),
        pl.BlockSpec((_R, _H), lambda i: (i, 0)),
        pl.BlockSpec((_R, _H // 2), lambda i: (i, 0)),
    ],
    out_shape=[
        jax.ShapeDtypeStruct((_N, _H), jnp.float32),
        jax.ShapeDtypeStruct((_N, _H), jnp.float32),
        jax.ShapeDtypeStruct((_N, _H // 2), jnp.int32),
    ],
)


def _tc_head_body(r2_ref, s2_ref, sq2_ref, g_ref, be_ref, x1_ref, x_ref,
                  wg1_ref, bg1_ref, wg2_ref, bg2_ref,
                  lw1_ref, lw2_ref, lwx_ref, linb_ref, y_ref):
    m = s2_ref[...] * (1.0 / _N)
    v = sq2_ref[...] * (1.0 / _N) - m * m
    scale = g_ref[...] * lax.rsqrt(v + 1e-5)
    x2 = (r2_ref[...] - m) * scale + be_ref[...]
    xc = jnp.concatenate([x1_ref[...], x2], axis=1)
    gt1 = jnp.dot(xc, wg1_ref[...], preferred_element_type=jnp.float32) + bg1_ref[...]
    i1 = jax.nn.sigmoid(gt1[:, 0:_H])
    gg1 = jnp.tanh(gt1[:, _H:2 * _H])
    o1 = jax.nn.sigmoid(gt1[:, 2 * _H:3 * _H])
    h1 = o1 * jnp.tanh(i1 * gg1)
    gt2 = jnp.dot(h1, wg2_ref[...], preferred_element_type=jnp.float32) + bg2_ref[...]
    i2 = jax.nn.sigmoid(gt2[:, 0:_H])
    gg2 = jnp.tanh(gt2[:, _H:2 * _H])
    o2 = jax.nn.sigmoid(gt2[:, 2 * _H:3 * _H])
    h2 = o2 * jnp.tanh(i2 * gg2)
    y = (jnp.dot(jnp.maximum(h1, 0.0), lw1_ref[...], preferred_element_type=jnp.float32)
         + jnp.dot(jnp.maximum(h2, 0.0), lw2_ref[...], preferred_element_type=jnp.float32)
         + jnp.dot(jnp.maximum(x_ref[...], 0.0), lwx_ref[...], preferred_element_type=jnp.float32)
         + linb_ref[...])
    y_ref[...] = y


_tc_head = pl.pallas_call(
    _tc_head_body,
    grid=(_G,),
    in_specs=[
        pl.BlockSpec((_R, _H), lambda i: (i, 0)),
        pl.BlockSpec((1, _H), lambda i: (0, 0)),
        pl.BlockSpec((1, _H), lambda i: (0, 0)),
        pl.BlockSpec((1, _H), lambda i: (0, 0)),
        pl.BlockSpec((1, _H), lambda i: (0, 0)),
        pl.BlockSpec((_R, _H), lambda i: (i, 0)),
        pl.BlockSpec((_R, _F), lambda i: (i, 0)),
        pl.BlockSpec((_F, 3 * _H), lambda i: (0, 0)),
        pl.BlockSpec((1, 3 * _H), lambda i: (0, 0)),
        pl.BlockSpec((_H, 3 * _H), lambda i: (0, 0)),
        pl.BlockSpec((1, 3 * _H), lambda i: (0, 0)),
        pl.BlockSpec((_H, 1), lambda i: (0, 0)),
        pl.BlockSpec((_H, 1), lambda i: (0, 0)),
        pl.BlockSpec((_F, 1), lambda i: (0, 0)),
        pl.BlockSpec((1, 1), lambda i: (0, 0)),
    ],
    out_specs=pl.BlockSpec((_R, 1), lambda i: (i, 0)),
    out_shape=jax.ShapeDtypeStruct((_N, 1), jnp.float32),
)


def kernel(x, edge_index, edge_weight, W1, b1, W2, b2, g1, be1, g2, be2,
           Wih1, Whh1, bih1, bhh1, Wih2, Whh2, bih2, bhh2, linW, linb):
    pad = _EP - _E
    src = jnp.pad(edge_index[0], (0, pad)).reshape(_NW, _K, _CH)
    dst = jnp.pad(edge_index[1], (0, pad)).reshape(_NW, _K, _CH)
    ewr = jnp.pad(edge_weight, (0, pad)).reshape(_NW, _K, _CH)

    degp = _sc_deg(dst, ewr)
    degT = degp.T[:_N]
    h1s, dinv, hb1 = _tc_a(x, W1, degT)

    acc1 = _sc_conv(hb1, src, dst, ewr)
    r1, s1, sq1 = _tc_stats(acc1, h1s, dinv, b1.reshape(1, _H))
    x1, h2s, hb2 = _tc_bn(r1, s1, sq1, g1.reshape(1, _H), be1.reshape(1, _H),
                          dinv, W2)

    acc2 = _sc_conv(hb2, src, dst, ewr)
    r2, s2, sq2 = _tc_stats(acc2, h2s, dinv, b2.reshape(1, _H))

    w1t = Wih1.T
    wg1 = jnp.concatenate([w1t[:, :_H], w1t[:, 2 * _H:]], axis=1)
    bgf1 = bih1 + bhh1
    bg1 = jnp.concatenate([bgf1[:_H], bgf1[2 * _H:]]).reshape(1, 3 * _H)
    w2t = Wih2.T
    wg2 = jnp.concatenate([w2t[:, :_H], w2t[:, 2 * _H:]], axis=1)
    bgf2 = bih2 + bhh2
    bg2 = jnp.concatenate([bgf2[:_H], bgf2[2 * _H:]]).reshape(1, 3 * _H)

    y = _tc_head(r2, s2, sq2, g2.reshape(1, _H), be2.reshape(1, _H), x1, x,
                 wg1, bg1, wg2, bg2,
                 linW[:_H], linW[_H:2 * _H], linW[2 * _H:],
                 linb.reshape(1, 1))
    return y


# speedup vs baseline: 1.0217x; 1.0217x over previous
import functools

import jax
import jax.numpy as jnp
from jax import lax
from jax.experimental import pallas as pl
from jax.experimental.pallas import tpu as pltpu
from jax.experimental.pallas import tpu_sc as plsc

_N = 10000
_F = 128
_H = 64
_E = 320000
_NC = 2
_NS = 16
_NW = _NC * _NS
_CH = 128
_K = 80
_EPT = _K * _CH
_EP = _NW * _EPT
_TS = 640
_NP = _NS * _TS
_R = 2000
_G = _N // _R

_mesh = plsc.VectorSubcoreMesh(core_axis_name="c", subcore_axis_name="s")


@functools.partial(
    pl.kernel,
    out_type=jax.ShapeDtypeStruct((_NC, _NP), jnp.float32),
    mesh=_mesh,
    compiler_params=pltpu.CompilerParams(use_tc_tiling_on_sc=False),
    scratch_types=[
        pltpu.VMEM((_K, _CH), jnp.int32),
        pltpu.VMEM((_K, _CH), jnp.float32),
        pltpu.VMEM((_TS,), jnp.float32),
        pltpu.VMEM_SHARED((_NP,), jnp.float32),
    ],
)
def _sc_deg(dst_hbm, ew_hbm, out_hbm, idx_v, ew_v, z_v, deg_sh):
    cid = lax.axis_index("c")
    sid = lax.axis_index("s")
    wid = sid * _NC + cid
    zero = jnp.zeros((16,), jnp.float32)
    for i in range(_TS // 16):
        z_v[pl.ds(i * 16, 16)] = zero
    pltpu.sync_copy(z_v, deg_sh.at[pl.ds(sid * _TS, _TS)])
    pltpu.sync_copy(dst_hbm.at[wid], idx_v)
    pltpu.sync_copy(ew_hbm.at[wid], ew_v)
    plsc.subcore_barrier()

    def body(j, carry):
        pltpu.sync_copy(ew_v.at[j], deg_sh.at[idx_v.at[j]], add=True)
        return carry

    lax.fori_loop(0, _K, body, 0)
    plsc.subcore_barrier()
    pltpu.sync_copy(deg_sh.at[pl.ds(sid * _TS, _TS)],
                    out_hbm.at[cid, pl.ds(sid * _TS, _TS)])


@functools.partial(
    pl.kernel,
    out_type=jax.ShapeDtypeStruct((_NC, _NP, _H), jnp.float32),
    mesh=_mesh,
    compiler_params=pltpu.CompilerParams(use_tc_tiling_on_sc=False,
                                         needs_layout_passes=False),
    scratch_types=[
        pltpu.VMEM((_K, _CH), jnp.int32),
        pltpu.VMEM((_K, _CH), jnp.int32),
        pltpu.VMEM((_K, _CH), jnp.float32),
        pltpu.VMEM((_CH, _H // 2), jnp.int32),
        pltpu.VMEM((_CH, _H // 2), jnp.int32),
        pltpu.VMEM((_CH, _H // 2), jnp.int32),
        pltpu.VMEM((_CH, _H // 2), jnp.int32),
        pltpu.VMEM((_CH, _H), jnp.float32),
        pltpu.VMEM((_CH, _H), jnp.float32),
        pltpu.VMEM_SHARED((_N, _H // 2), jnp.int32),
        pltpu.VMEM_SHARED((_NP, _H), jnp.float32),
        pltpu.SemaphoreType.DMA,
        pltpu.SemaphoreType.DMA,
        pltpu.SemaphoreType.DMA,
        pltpu.SemaphoreType.DMA,
        pltpu.SemaphoreType.DMA,
        pltpu.SemaphoreType.DMA,
    ],
)
def _sc_conv(h_hbm, src_hbm, dst_hbm, ew_hbm, out_hbm,
             src_v, dst_v, ew_v,
             gb0, gb1, gb2, gb3,
             sb0, sb1,
             h_sh, acc_sh,
             g0, g1, g2, g3, s0, s1):
    cid = lax.axis_index("c")
    sid = lax.axis_index("s")
    wid = sid * _NC + cid
    zero = jnp.zeros((16,), jnp.float32)
    for i in range(_CH):
        for f in range(_H // 16):
            sb0[i, pl.ds(f * 16, 16)] = zero
    for t in range(_TS // _CH):
        pltpu.sync_copy(sb0, acc_sh.at[pl.ds(sid * _TS + t * _CH, _CH)])
    pltpu.sync_copy(h_hbm.at[pl.ds(sid * (_N // _NS), _N // _NS)],
                    h_sh.at[pl.ds(sid * (_N // _NS), _N // _NS)])
    pltpu.sync_copy(src_hbm.at[wid], src_v)
    pltpu.sync_copy(dst_hbm.at[wid], dst_v)
    pltpu.sync_copy(ew_hbm.at[wid], ew_v)
    plsc.subcore_barrier()

    _MASK = jnp.int32(-65536)

    def _mult(j, gb, sb):
        def edge_group(g, c2):
            wv = ew_v[j, pl.ds(g * 16, 16)]
            for l in range(16):
                e = g * 16 + l
                w = wv[l]
                for f in range(2):
                    wi = gb[e, pl.ds(f * 16, 16)]
                    ev = plsc.bitcast(wi << 16, jnp.float32)
                    od = plsc.bitcast(wi & _MASK, jnp.float32)
                    sb[e, pl.ds(f * 32, 16)] = ev * w
                    sb[e, pl.ds(f * 32 + 16, 16)] = od * w
            return c2
        lax.fori_loop(0, _CH // 16, edge_group, 0, unroll=2)

    def _g_start(j, gb, sem):
        pltpu.async_copy(h_sh.at[src_v.at[j]], gb, sem)

    def _g_wait(gb, sem):
        pltpu.make_async_copy(h_sh.at[src_v.at[0]], gb, sem).wait()

    def _s_start(j, sb, sem):
        pltpu.async_copy(sb, acc_sh.at[dst_v.at[j]], sem, add=True)

    def _s_wait(sb, sem):
        pltpu.make_async_copy(sb, acc_sh.at[dst_v.at[0]], sem).wait()

    gbufs = (gb0, gb1, gb2, gb3)
    sbufs = (sb0, sb1)
    gs = (g0, g1, g2, g3)
    ss = (s0, s1)
    _NG = 4
    _SBI = (0, 1, 0, 1)
    _SDIST = (2, 2, 2, 2)

    for b in range(_NG - 1):
        _g_start(b, gbufs[b], gs[b])

    def octet(i, carry):
        for b in range(_NG):
            c = i + b
            _g_wait(gbufs[b], gs[b])
            nb = (b + _NG - 1) % _NG

            @pl.when(c + _NG - 1 < _K)
            def _(c=c, nb=nb):
                _g_start(c + _NG - 1, gbufs[nb], gs[nb])

            sbi = _SBI[b]

            @pl.when(c >= _SDIST[b])
            def _(sbi=sbi):
                _s_wait(sbufs[sbi], ss[sbi])

            _mult(c, gbufs[b], sbufs[sbi])
            _s_start(c, sbufs[sbi], ss[sbi])
        return carry

    lax.fori_loop(0, _K // _NG, lambda i, c: octet(i * _NG, c), 0)
    for b in range(2):
        _s_wait(sbufs[b], ss[b])
    plsc.subcore_barrier()
    pltpu.sync_copy(acc_sh.at[pl.ds(sid * _TS, _TS)],
                    out_hbm.at[cid, pl.ds(sid * _TS, _TS)])


def _pack_rows(h, hb_ref):
    for g in range(2):
        lo = lax.bitcast_convert_type(
            h[:, 32 * g:32 * g + 16].astype(jnp.bfloat16), jnp.int16)
        hi = lax.bitcast_convert_type(
            h[:, 32 * g + 16:32 * g + 32].astype(jnp.bfloat16), jnp.int16)
        word = (lo.astype(jnp.int32) & 0xFFFF) | (hi.astype(jnp.int32) << 16)
        hb_ref[:, 16 * g:16 * g + 16] = word


def _tc_a_body(x_ref, w_ref, degT_ref, h1s_ref, dinv_ref, hb_ref):
    deg = degT_ref[:, 0:1] + degT_ref[:, 1:2] + 1.0
    dinv = jnp.where(deg > 0, lax.rsqrt(jnp.maximum(deg, 1e-30)), 0.0)
    h = jnp.dot(x_ref[...], w_ref[...], preferred_element_type=jnp.float32)
    h1s = h * dinv
    h1s_ref[...] = h1s
    dinv_ref[...] = dinv
    _pack_rows(h1s, hb_ref)


_tc_a = pl.pallas_call(
    _tc_a_body,
    grid=(_G,),
    in_specs=[
        pl.BlockSpec((_R, _F), lambda i: (i, 0)),
        pl.BlockSpec((_F, _H), lambda i: (0, 0)),
        pl.BlockSpec((_R, 2), lambda i: (i, 0)),
    ],
    out_specs=[
        pl.BlockSpec((_R, _H), lambda i: (i, 0)),
        pl.BlockSpec((_R, 1), lambda i: (i, 0)),
        pl.BlockSpec((_R, _H // 2), lambda i: (i, 0)),
    ],
    out_shape=[
        jax.ShapeDtypeStruct((_N, _H), jnp.float32),
        jax.ShapeDtypeStruct((_N, 1), jnp.float32),
        jax.ShapeDtypeStruct((_N, _H // 2), jnp.int32),
    ],
)


def _tc_stats_body(acc_ref, hs_ref, dinv_ref, b_ref, r_ref, s_ref, sq_ref):
    a = acc_ref[0] + acc_ref[1] + hs_ref[...]
    r = jnp.maximum(b_ref[...] + dinv_ref[...] * a, 0.0)
    r_ref[...] = r

    @pl.when(pl.program_id(0) == 0)
    def _():
        s_ref[...] = jnp.zeros_like(s_ref)
        sq_ref[...] = jnp.zeros_like(sq_ref)

    s_ref[...] += jnp.sum(r, axis=0, keepdims=True)
    sq_ref[...] += jnp.sum(r * r, axis=0, keepdims=True)


_tc_stats = pl.pallas_call(
    _tc_stats_body,
    grid=(_G,),
    in_specs=[
        pl.BlockSpec((_NC, _R, _H), lambda i: (0, i, 0)),
        pl.BlockSpec((_R, _H), lambda i: (i, 0)),
        pl.BlockSpec((_R, 1), lambda i: (i, 0)),
        pl.BlockSpec((1, _H), lambda i: (0, 0)),
    ],
    out_specs=[
        pl.BlockSpec((_R, _H), lambda i: (i, 0)),
        pl.BlockSpec((1, _H), lambda i: (0, 0)),
        pl.BlockSpec((1, _H), lambda i: (0, 0)),
    ],
    out_shape=[
        jax.ShapeDtypeStruct((_N, _H), jnp.float32),
        jax.ShapeDtypeStruct((1, _H), jnp.float32),
        jax.ShapeDtypeStruct((1, _H), jnp.float32),
    ],
)


def _tc_bn_body(r_ref, s_ref, sq_ref, g_ref, be_ref, dinv_ref, w_ref,
                x1_ref, hs2_ref, hb_ref):
    m = s_ref[...] * (1.0 / _N)
    v = sq_ref[...] * (1.0 / _N) - m * m
    scale = g_ref[...] * lax.rsqrt(v + 1e-5)
    x1 = (r_ref[...] - m) * scale + be_ref[...]
    x1_ref[...] = x1
    hs2 = jnp.dot(x1, w_ref[...],
                  preferred_element_type=jnp.float32) * dinv_ref[...]
    hs2_ref[...] = hs2
    _pack_rows(hs2, hb_ref)


_tc_bn = pl.pallas_call(
    _tc_bn_body,
    grid=(_G,),
    in_specs=[
        pl.BlockSpec((_R, _H), lambda i: (i, 0)),
        pl.BlockSpec((1, _H), lambda i: (0, 0)),
        pl.BlockSpec((1, _H), lambda i: (0, 0)),
        pl.BlockSpec((1, _H), lambda i: (0, 0)),
        pl.BlockSpec((1, _H), lambda i: (0, 0)),
        pl.BlockSpec((_R, 1), lambda i: (i, 0)),
        pl.BlockSpec((_H, _H), lambda i: (0, 0)),
    ],
    out_specs=[
        pl.BlockSpec((_R, _H), lambda i: (i, 0)),
        pl.BlockSpec((_R, _H), lambda i: (i, 0)),
        pl.BlockSpec((_R, _H // 2), lambda i: (i, 0)),
    ],
    out_shape=[
        jax.ShapeDtypeStruct((_N, _H), jnp.float32),
        jax.ShapeDtypeStruct((_N, _H), jnp.float32),
        jax.ShapeDtypeStruct((_N, _H // 2), jnp.int32),
    ],
)


def _tc_head_body(r2_ref, s2_ref, sq2_ref, g_ref, be_ref, x1_ref, x_ref,
                  wg1_ref, bg1_ref, wg2_ref, bg2_ref,
                  lw1_ref, lw2_ref, lwx_ref, linb_ref, y_ref):
    m = s2_ref[...] * (1.0 / _N)
    v = sq2_ref[...] * (1.0 / _N) - m * m
    scale = g_ref[...] * lax.rsqrt(v + 1e-5)
    x2 = (r2_ref[...] - m) * scale + be_ref[...]
    xc = jnp.concatenate([x1_ref[...], x2], axis=1)
    gt1 = jnp.dot(xc, wg1_ref[...], preferred_element_type=jnp.float32) + bg1_ref[...]
    i1 = jax.nn.sigmoid(gt1[:, 0:_H])
    gg1 = jnp.tanh(gt1[:, _H:2 * _H])
    o1 = jax.nn.sigmoid(gt1[:, 2 * _H:3 * _H])
    h1 = o1 * jnp.tanh(i1 * gg1)
    gt2 = jnp.dot(h1, wg2_ref[...], preferred_element_type=jnp.float32) + bg2_ref[...]
    i2 = jax.nn.sigmoid(gt2[:, 0:_H])
    gg2 = jnp.tanh(gt2[:, _H:2 * _H])
    o2 = jax.nn.sigmoid(gt2[:, 2 * _H:3 * _H])
    h2 = o2 * jnp.tanh(i2 * gg2)
    y = (jnp.dot(jnp.maximum(h1, 0.0), lw1_ref[...], preferred_element_type=jnp.float32)
         + jnp.dot(jnp.maximum(h2, 0.0), lw2_ref[...], preferred_element_type=jnp.float32)
         + jnp.dot(jnp.maximum(x_ref[...], 0.0), lwx_ref[...], preferred_element_type=jnp.float32)
         + linb_ref[...])
    y_ref[...] = y


_tc_head = pl.pallas_call(
    _tc_head_body,
    grid=(_G,),
    in_specs=[
        pl.BlockSpec((_R, _H), lambda i: (i, 0)),
        pl.BlockSpec((1, _H), lambda i: (0, 0)),
        pl.BlockSpec((1, _H), lambda i: (0, 0)),
        pl.BlockSpec((1, _H), lambda i: (0, 0)),
        pl.BlockSpec((1, _H), lambda i: (0, 0)),
        pl.BlockSpec((_R, _H), lambda i: (i, 0)),
        pl.BlockSpec((_R, _F), lambda i: (i, 0)),
        pl.BlockSpec((_F, 3 * _H), lambda i: (0, 0)),
        pl.BlockSpec((1, 3 * _H), lambda i: (0, 0)),
        pl.BlockSpec((_H, 3 * _H), lambda i: (0, 0)),
        pl.BlockSpec((1, 3 * _H), lambda i: (0, 0)),
        pl.BlockSpec((_H, 1), lambda i: (0, 0)),
        pl.BlockSpec((_H, 1), lambda i: (0, 0)),
        pl.BlockSpec((_F, 1), lambda i: (0, 0)),
        pl.BlockSpec((1, 1), lambda i: (0, 0)),
    ],
    out_specs=pl.BlockSpec((_R, 1), lambda i: (i, 0)),
    out_shape=jax.ShapeDtypeStruct((_N, 1), jnp.float32),
)


def kernel(x, edge_index, edge_weight, W1, b1, W2, b2, g1, be1, g2, be2,
           Wih1, Whh1, bih1, bhh1, Wih2, Whh2, bih2, bhh2, linW, linb):
    pad = _EP - _E
    src = jnp.pad(edge_index[0], (0, pad)).reshape(_NW, _K, _CH)
    dst = jnp.pad(edge_index[1], (0, pad)).reshape(_NW, _K, _CH)
    ewr = jnp.pad(edge_weight, (0, pad)).reshape(_NW, _K, _CH)

    degp = _sc_deg(dst, ewr)
    degT = degp.T[:_N]
    h1s, dinv, hb1 = _tc_a(x, W1, degT)

    acc1 = _sc_conv(hb1, src, dst, ewr)
    r1, s1, sq1 = _tc_stats(acc1, h1s, dinv, b1.reshape(1, _H))
    x1, h2s, hb2 = _tc_bn(r1, s1, sq1, g1.reshape(1, _H), be1.reshape(1, _H),
                          dinv, W2)

    acc2 = _sc_conv(hb2, src, dst, ewr)
    r2, s2, sq2 = _tc_stats(acc2, h2s, dinv, b2.reshape(1, _H))

    w1t = Wih1.T
    wg1 = jnp.concatenate([w1t[:, :_H], w1t[:, 2 * _H:]], axis=1)
    bgf1 = bih1 + bhh1
    bg1 = jnp.concatenate([bgf1[:_H], bgf1[2 * _H:]]).reshape(1, 3 * _H)
    w2t = Wih2.T
    wg2 = jnp.concatenate([w2t[:, :_H], w2t[:, 2 * _H:]], axis=1)
    bgf2 = bih2 + bhh2
    bg2 = jnp.concatenate([bgf2[:_H], bgf2[2 * _H:]]).reshape(1, 3 * _H)

    y = _tc_head(r2, s2, sq2, g2.reshape(1, _H), be2.reshape(1, _H), x1, x,
                 wg1, bg1, wg2, bg2,
                 linW[:_H], linW[_H:2 * _H], linW[2 * _H:],
                 linb.reshape(1, 1))
    return y
